# baseline (speedup 1.0000x reference)
import jax
import jax.numpy as jnp
from jax import lax
from jax.experimental import pallas as pl
from jax.experimental.pallas import tpu as pltpu
from jax.experimental.pallas import tpu_sc as plsc

B = 16384
NC, NS = 2, 16
NW = NC * NS
BPW = B // NW
GRP = 16
NG = BPW // GRP
CHUNK = 2048
NCHUNK = B // CHUNK
EPS = 1e-5


def _sc_gather_body(t_user, t_country, t_device, t_interest,
                    i_user, i_country, i_device, i_interest,
                    o_user, o_country, o_device, o_interest,
                    iv, gb, ob32, ob16, sem):
    wid = lax.axis_index("s") * NC + lax.axis_index("c")
    base = wid * BPW
    i32 = jnp.int32

    def do_table(tT, ih, gb, ob, d, oh):
        pltpu.sync_copy(ih.at[pl.ds(base, BPW)], iv)

        def grp(g, carry):
            j0 = g * GRP
            v16 = iv[pl.ds(j0, GRP)]
            c16 = lax.bitwise_and(v16, jnp.int32(-128))
            cps = []
            for k in range(GRP):
                c0 = pl.multiple_of(c16[k], 128)
                cps.append(pltpu.async_copy(
                    tT.at[pl.ds(0, d), pl.ds(c0, 128)],
                    gb.at[pl.ds(0, d), pl.ds(k * 128, 128)], sem))
            for cp in cps:
                cp.wait()
            sub16 = lax.bitwise_and(v16, 127)
            lane16 = lax.iota(i32, GRP) * 128 + sub16
            col16 = lax.iota(i32, GRP) + j0
            for f in range(d):
                f16 = jnp.full((GRP,), f, i32)
                vals = plsc.load_gather(gb, [f16, lane16])
                plsc.store_scatter(ob, [f16, col16], vals)
            return carry

        lax.fori_loop(0, NG, grp, 0)
        pltpu.sync_copy(ob, oh.at[pl.ds(0, d), pl.ds(base, BPW)])

    do_table(t_user, i_user, gb, ob32, 32, o_user)
    do_table(t_interest, i_interest, gb, ob32, 32, o_interest)
    do_table(t_country, i_country, gb, ob16, 16, o_country)
    do_table(t_device, i_device, gb, ob16, 16, o_device)


def _sc_gather(tables, idxs):
    mesh = plsc.VectorSubcoreMesh(core_axis_name="c", subcore_axis_name="s")
    out_type = [jax.ShapeDtypeStruct((32, B), jnp.float32),
                jax.ShapeDtypeStruct((16, B), jnp.float32),
                jax.ShapeDtypeStruct((16, B), jnp.float32),
                jax.ShapeDtypeStruct((32, B), jnp.float32)]
    scratch = [pltpu.VMEM((BPW,), jnp.int32),
               pltpu.VMEM((32, GRP * 128), jnp.float32),
               pltpu.VMEM((32, BPW), jnp.float32),
               pltpu.VMEM((16, BPW), jnp.float32),
               pltpu.SemaphoreType.DMA]
    fn = pl.kernel(_sc_gather_body, out_type=out_type, mesh=mesh,
                   scratch_types=scratch,
                   compiler_params=pltpu.CompilerParams(
                       needs_layout_passes=False))
    return fn(*tables, *idxs)


def _mlp_body(euT, ecT, edT, eiT, nmT,
              W1r, b1r, g1r, bt1r, W2r, b2r, g2r, bt2r, W3r, b3r,
              out_ref, h1s, h2s, s1, q1, s2, q2):
    f32 = jnp.float32
    p = pl.program_id(0)
    c = pl.program_id(1)
    sl = pl.ds(c * CHUNK, CHUNK)
    dn = (((0,), (0,)), ((), ()))

    @pl.when(p == 0)
    def _phase0():
        h = (lax.dot_general(euT[...], W1r[0:32, :], dn,
                             preferred_element_type=f32)
             + lax.dot_general(ecT[...], W1r[32:48, :], dn,
                               preferred_element_type=f32)
             + lax.dot_general(edT[...], W1r[48:64, :], dn,
                               preferred_element_type=f32)
             + lax.dot_general(eiT[...], W1r[64:96, :], dn,
                               preferred_element_type=f32)
             + lax.dot_general(nmT[...], W1r[96:112, :], dn,
                               preferred_element_type=f32)
             + b1r[...])
        h = jnp.maximum(h, 0.0)
        h1s[sl, :] = h
        hs = jnp.sum(h, axis=0, keepdims=True)
        hq = jnp.sum(h * h, axis=0, keepdims=True)

        @pl.when(c == 0)
        def _():
            s1[...] = hs
            q1[...] = hq

        @pl.when(c != 0)
        def _():
            s1[...] += hs
            q1[...] += hq

    @pl.when(p == 1)
    def _phase1():
        m1 = s1[...] * (1.0 / B)
        v1 = jnp.maximum(q1[...] * (1.0 / B) - m1 * m1, 0.0)
        a1 = lax.rsqrt(v1 + EPS) * g1r[...]
        c1 = bt1r[...] - m1 * a1
        hn = h1s[sl, :] * a1 + c1
        h2 = jnp.maximum(jnp.dot(hn, W2r[...], preferred_element_type=f32)
                         + b2r[...], 0.0)
        h2s[sl, :] = h2
        hs = jnp.sum(h2, axis=0, keepdims=True)
        hq = jnp.sum(h2 * h2, axis=0, keepdims=True)

        @pl.when(c == 0)
        def _():
            s2[...] = hs
            q2[...] = hq

        @pl.when(c != 0)
        def _():
            s2[...] += hs
            q2[...] += hq

    @pl.when(p == 2)
    def _phase2():
        m2 = s2[...] * (1.0 / B)
        v2 = jnp.maximum(q2[...] * (1.0 / B) - m2 * m2, 0.0)
        a2 = lax.rsqrt(v2 + EPS) * g2r[...]
        c2 = bt2r[...] - m2 * a2
        hn = h2s[sl, :] * a2 + c2
        out_ref[...] = (jnp.dot(hn, W3r[...], preferred_element_type=f32)
                        + b3r[...])


def _mlp_tc(euT, ecT, edT, eiT, nmT, W1, b1, g1, bt1, W2, b2, g2, bt2,
            W3, b3):
    def chunk_specT(d):
        return pl.BlockSpec((d, CHUNK),
                            lambda p, c: (0, jnp.where(p == 0, c, 0)))

    def full_spec(shape):
        return pl.BlockSpec(shape, lambda p, c: (0, 0))

    return pl.pallas_call(
        _mlp_body,
        grid=(3, NCHUNK),
        in_specs=[
            chunk_specT(32), chunk_specT(16), chunk_specT(16),
            chunk_specT(32), chunk_specT(16),
            full_spec((112, 256)), full_spec((1, 256)), full_spec((1, 256)),
            full_spec((1, 256)),
            full_spec((256, 128)), full_spec((1, 128)), full_spec((1, 128)),
            full_spec((1, 128)),
            full_spec((128, 64)), full_spec((1, 64)),
        ],
        out_specs=pl.BlockSpec((CHUNK, 64),
                               lambda p, c: (jnp.where(p == 2, c, 0), 0)),
        out_shape=jax.ShapeDtypeStruct((B, 64), jnp.float32),
        scratch_shapes=[
            pltpu.VMEM((B, 256), jnp.float32),
            pltpu.VMEM((B, 128), jnp.float32),
            pltpu.VMEM((1, 256), jnp.float32),
            pltpu.VMEM((1, 256), jnp.float32),
            pltpu.VMEM((1, 128), jnp.float32),
            pltpu.VMEM((1, 128), jnp.float32),
        ],
    )(euT, ecT, edT, eiT, nmT, W1, b1, g1, bt1, W2, b2, g2, bt2, W3, b3)


def kernel(cat_user_id, cat_country, cat_device, cat_interest,
           numerical_inputs,
           T_user, T_country, T_device, T_interest,
           W1, b1, g1, bt1, W2, b2, g2, bt2, W3, b3):
    idxs = (cat_user_id[:, 0], cat_country[:, 0], cat_device[:, 0],
            cat_interest[:, 0])
    tablesT = (T_user.T, T_country.T, T_device.T, T_interest.T)
    euT, ecT, edT, eiT = _sc_gather(tablesT, idxs)
    return _mlp_tc(euT, ecT, edT, eiT, numerical_inputs.T,
                   W1, b1.reshape(1, 256), g1.reshape(1, 256),
                   bt1.reshape(1, 256), W2, b2.reshape(1, 128),
                   g2.reshape(1, 128), bt2.reshape(1, 128),
                   W3, b3.reshape(1, 64))


# speedup vs baseline: 2.6722x; 2.6722x over previous
import jax
import jax.numpy as jnp
from jax import lax
from jax.experimental import pallas as pl
from jax.experimental.pallas import tpu as pltpu
from jax.experimental.pallas import tpu_sc as plsc

B = 16384
NC, NS = 2, 16
NW = NC * NS
BPW = B // NW
GRP = 16
NG = BPW // GRP
CHUNK = 2048
NCHUNK = B // CHUNK
EPS = 1e-5


def _sc_gather_body(t_user, t_country, t_device, t_interest,
                    i_user, i_country, i_device, i_interest,
                    o_user, o_country, o_device, o_interest,
                    ivu, ivx, pv, gbu, gbi, ob32, ob16, sem, semi):
    wid = lax.axis_index("s") * NC + lax.axis_index("c")
    base = wid * BPW
    i32 = jnp.int32
    HP = BPW // 2

    def stage_packed(ih, shift):
        pltpu.sync_copy(ih.at[pl.ds(base, BPW)], ivx)
        for j in range(BPW // 16):
            pv[pl.ds(j * 16, 16)] = lax.shift_right_logical(
                ivx[pl.ds(j * 16, 16)], shift)

    def xtract_packed(d, mask, half, ob):
        def grp(g, carry):
            jr = g * GRP
            v16 = ivx[pl.ds(half * HP + jr, GRP)]
            sub16 = lax.bitwise_and(v16, mask)
            row16 = lax.iota(i32, GRP) + jr
            col16 = lax.iota(i32, GRP) + (half * HP + jr)
            for f in range(d):
                f16 = jnp.full((GRP,), f, i32)
                vals = plsc.load_gather(gbi, [row16, sub16 * d + f])
                plsc.store_scatter(ob, [f16, col16], vals)
            return carry

        lax.fori_loop(0, HP // GRP, grp, 0)

    def do_packed(tp, d, mask, ob, oh):
        cp = pltpu.async_copy(tp.at[pv.at[pl.ds(0, HP)]], gbi, semi)
        cp.wait()
        xtract_packed(d, mask, 0, ob)
        cp = pltpu.async_copy(tp.at[pv.at[pl.ds(HP, HP)]], gbi, semi)
        cp.wait()
        xtract_packed(d, mask, 1, ob)
        pltpu.sync_copy(ob, oh.at[pl.ds(0, d), pl.ds(base, BPW)])

    stage_packed(i_interest, 2)
    cp_i1 = pltpu.async_copy(t_interest.at[pv.at[pl.ds(0, HP)]], gbi, semi)

    pltpu.sync_copy(i_user.at[pl.ds(base, BPW)], ivu)

    def ugrp(g, carry):
        j0 = g * GRP
        v16 = ivu[pl.ds(j0, GRP)]
        c16 = lax.bitwise_and(v16, jnp.int32(-128))
        cps = []
        for k in range(GRP):
            c0 = pl.multiple_of(c16[k], 128)
            cps.append(pltpu.async_copy(
                t_user.at[pl.ds(0, 32), pl.ds(c0, 128)],
                gbu.at[pl.ds(0, 32), pl.ds(k * 128, 128)], sem))
        for cp in cps:
            cp.wait()
        sub16 = lax.bitwise_and(v16, 127)
        lane16 = lax.iota(i32, GRP) * 128 + sub16
        col16 = lax.iota(i32, GRP) + j0
        for f in range(32):
            f16 = jnp.full((GRP,), f, i32)
            vals = plsc.load_gather(gbu, [f16, lane16])
            plsc.store_scatter(ob32, [f16, col16], vals)
        return carry

    lax.fori_loop(0, NG, ugrp, 0)
    pltpu.sync_copy(ob32, o_user.at[pl.ds(0, 32), pl.ds(base, BPW)])

    cp_i1.wait()
    xtract_packed(32, 3, 0, ob32)
    cp_i2 = pltpu.async_copy(t_interest.at[pv.at[pl.ds(HP, HP)]], gbi, semi)
    cp_i2.wait()
    xtract_packed(32, 3, 1, ob32)
    pltpu.sync_copy(ob32, o_interest.at[pl.ds(0, 32), pl.ds(base, BPW)])

    stage_packed(i_country, 3)
    do_packed(t_country, 16, 7, ob16, o_country)
    stage_packed(i_device, 3)
    do_packed(t_device, 16, 7, ob16, o_device)


def _sc_gather(tables, idxs):
    mesh = plsc.VectorSubcoreMesh(core_axis_name="c", subcore_axis_name="s")
    out_type = [jax.ShapeDtypeStruct((32, B), jnp.float32),
                jax.ShapeDtypeStruct((16, B), jnp.float32),
                jax.ShapeDtypeStruct((16, B), jnp.float32),
                jax.ShapeDtypeStruct((32, B), jnp.float32)]
    scratch = [pltpu.VMEM((BPW,), jnp.int32),
               pltpu.VMEM((BPW,), jnp.int32),
               pltpu.VMEM((BPW,), jnp.int32),
               pltpu.VMEM((32, GRP * 128), jnp.float32),
               pltpu.VMEM((BPW // 2, 128), jnp.float32),
               pltpu.VMEM((32, BPW), jnp.float32),
               pltpu.VMEM((16, BPW), jnp.float32),
               pltpu.SemaphoreType.DMA,
               pltpu.SemaphoreType.DMA]
    fn = pl.kernel(_sc_gather_body, out_type=out_type, mesh=mesh,
                   scratch_types=scratch,
                   compiler_params=pltpu.CompilerParams(
                       needs_layout_passes=False))
    return fn(*tables, *idxs)


def _mlp_body(euT, ecT, edT, eiT, nmT,
              W1r, b1r, g1r, bt1r, W2r, b2r, g2r, bt2r, W3r, b3r,
              out_ref, h1s, h2s, s1, q1, s2, q2):
    f32 = jnp.float32
    p = pl.program_id(0)
    c = pl.program_id(1)
    sl = pl.ds(c * CHUNK, CHUNK)
    dn = (((0,), (0,)), ((), ()))

    @pl.when(p == 0)
    def _phase0():
        h = (lax.dot_general(euT[...], W1r[0:32, :], dn,
                             preferred_element_type=f32)
             + lax.dot_general(ecT[...], W1r[32:48, :], dn,
                               preferred_element_type=f32)
             + lax.dot_general(edT[...], W1r[48:64, :], dn,
                               preferred_element_type=f32)
             + lax.dot_general(eiT[...], W1r[64:96, :], dn,
                               preferred_element_type=f32)
             + lax.dot_general(nmT[...], W1r[96:112, :], dn,
                               preferred_element_type=f32)
             + b1r[...])
        h = jnp.maximum(h, 0.0)
        h1s[sl, :] = h
        hs = jnp.sum(h, axis=0, keepdims=True)
        hq = jnp.sum(h * h, axis=0, keepdims=True)

        @pl.when(c == 0)
        def _():
            s1[...] = hs
            q1[...] = hq

        @pl.when(c != 0)
        def _():
            s1[...] += hs
            q1[...] += hq

    @pl.when(p == 1)
    def _phase1():
        m1 = s1[...] * (1.0 / B)
        v1 = jnp.maximum(q1[...] * (1.0 / B) - m1 * m1, 0.0)
        a1 = lax.rsqrt(v1 + EPS) * g1r[...]
        c1 = bt1r[...] - m1 * a1
        hn = h1s[sl, :] * a1 + c1
        h2 = jnp.maximum(jnp.dot(hn, W2r[...], preferred_element_type=f32)
                         + b2r[...], 0.0)
        h2s[sl, :] = h2
        hs = jnp.sum(h2, axis=0, keepdims=True)
        hq = jnp.sum(h2 * h2, axis=0, keepdims=True)

        @pl.when(c == 0)
        def _():
            s2[...] = hs
            q2[...] = hq

        @pl.when(c != 0)
        def _():
            s2[...] += hs
            q2[...] += hq

    @pl.when(p == 2)
    def _phase2():
        m2 = s2[...] * (1.0 / B)
        v2 = jnp.maximum(q2[...] * (1.0 / B) - m2 * m2, 0.0)
        a2 = lax.rsqrt(v2 + EPS) * g2r[...]
        c2 = bt2r[...] - m2 * a2
        hn = h2s[sl, :] * a2 + c2
        out_ref[...] = (jnp.dot(hn, W3r[...], preferred_element_type=f32)
                        + b3r[...])


def _mlp_tc(euT, ecT, edT, eiT, nmT, W1, b1, g1, bt1, W2, b2, g2, bt2,
            W3, b3):
    def chunk_specT(d):
        return pl.BlockSpec((d, CHUNK),
                            lambda p, c: (0, jnp.where(p == 0, c, 0)))

    def full_spec(shape):
        return pl.BlockSpec(shape, lambda p, c: (0, 0))

    return pl.pallas_call(
        _mlp_body,
        grid=(3, NCHUNK),
        in_specs=[
            chunk_specT(32), chunk_specT(16), chunk_specT(16),
            chunk_specT(32), chunk_specT(16),
            full_spec((112, 256)), full_spec((1, 256)), full_spec((1, 256)),
            full_spec((1, 256)),
            full_spec((256, 128)), full_spec((1, 128)), full_spec((1, 128)),
            full_spec((1, 128)),
            full_spec((128, 64)), full_spec((1, 64)),
        ],
        out_specs=pl.BlockSpec((CHUNK, 64),
                               lambda p, c: (jnp.where(p == 2, c, 0), 0)),
        out_shape=jax.ShapeDtypeStruct((B, 64), jnp.float32),
        scratch_shapes=[
            pltpu.VMEM((B, 256), jnp.float32),
            pltpu.VMEM((B, 128), jnp.float32),
            pltpu.VMEM((1, 256), jnp.float32),
            pltpu.VMEM((1, 256), jnp.float32),
            pltpu.VMEM((1, 128), jnp.float32),
            pltpu.VMEM((1, 128), jnp.float32),
        ],
    )(euT, ecT, edT, eiT, nmT, W1, b1, g1, bt1, W2, b2, g2, bt2, W3, b3)


def kernel(cat_user_id, cat_country, cat_device, cat_interest,
           numerical_inputs,
           T_user, T_country, T_device, T_interest,
           W1, b1, g1, bt1, W2, b2, g2, bt2, W3, b3):
    idxs = (cat_user_id[:, 0], cat_country[:, 0], cat_device[:, 0],
            cat_interest[:, 0])
    tables = (T_user.T, T_country.reshape(-1, 128),
              T_device.reshape(-1, 128), T_interest.reshape(-1, 128))
    euT, ecT, edT, eiT = _sc_gather(tables, idxs)
    return _mlp_tc(euT, ecT, edT, eiT, numerical_inputs.T,
                   W1, b1.reshape(1, 256), g1.reshape(1, 256),
                   bt1.reshape(1, 256), W2, b2.reshape(1, 128),
                   g2.reshape(1, 128), bt2.reshape(1, 128),
                   W3, b3.reshape(1, 64))


# speedup vs baseline: 2.7803x; 1.0405x over previous
import jax
import jax.numpy as jnp
from jax import lax
from jax.experimental import pallas as pl
from jax.experimental.pallas import tpu as pltpu
from jax.experimental.pallas import tpu_sc as plsc

B = 16384
NC, NS = 2, 16
NW = NC * NS
BPW = B // NW
GRP = 16
NG = BPW // GRP
CHUNK = 2048
NCHUNK = B // CHUNK
EPS = 1e-5


def _sc_user_body(t_user, i_user, o_user, ivu, gbu, ob32, sem):
    wid = lax.axis_index("s") * NC + lax.axis_index("c")
    base = wid * BPW
    i32 = jnp.int32
    pltpu.sync_copy(i_user.at[pl.ds(base, BPW)], ivu)

    def ugrp(g, carry):
        j0 = g * GRP
        v16 = ivu[pl.ds(j0, GRP)]
        c16 = lax.bitwise_and(v16, jnp.int32(-128))
        cps = []
        for k in range(GRP):
            c0 = pl.multiple_of(c16[k], 128)
            cps.append(pltpu.async_copy(
                t_user.at[pl.ds(0, 32), pl.ds(c0, 128)],
                gbu.at[pl.ds(0, 32), pl.ds(k * 128, 128)], sem))
        for cp in cps:
            cp.wait()
        sub16 = lax.bitwise_and(v16, 127)
        lane16 = lax.iota(i32, GRP) * 128 + sub16
        col16 = lax.iota(i32, GRP) + j0
        for f in range(32):
            f16 = jnp.full((GRP,), f, i32)
            vals = plsc.load_gather(gbu, [f16, lane16])
            plsc.store_scatter(ob32, [f16, col16], vals)
        return carry

    lax.fori_loop(0, NG, ugrp, 0)
    pltpu.sync_copy(ob32, o_user.at[pl.ds(0, 32), pl.ds(base, BPW)])


def _sc_packed_body(t_country, t_device, t_interest,
                    i_country, i_device, i_interest,
                    o_country, o_device, o_interest,
                    ivx, pv, gbi, ob32, ob16, semi):
    wid = lax.axis_index("s") * NC + lax.axis_index("c")
    base = wid * BPW
    i32 = jnp.int32

    def stage_packed(ih, shift):
        pltpu.sync_copy(ih.at[pl.ds(base, BPW)], ivx)
        for j in range(BPW // 16):
            pv[pl.ds(j * 16, 16)] = lax.shift_right_logical(
                ivx[pl.ds(j * 16, 16)], shift)

    def do_packed(tp, d, mask, ob, oh):
        cp = pltpu.async_copy(tp.at[pv], gbi, semi)
        cp.wait()

        def grp(g, carry):
            jr = g * GRP
            v16 = ivx[pl.ds(jr, GRP)]
            sub16 = lax.bitwise_and(v16, mask)
            row16 = lax.iota(i32, GRP) + jr
            col16 = lax.iota(i32, GRP) + jr
            for f in range(d):
                f16 = jnp.full((GRP,), f, i32)
                vals = plsc.load_gather(gbi, [row16, sub16 * d + f])
                plsc.store_scatter(ob, [f16, col16], vals)
            return carry

        lax.fori_loop(0, NG, grp, 0)
        pltpu.sync_copy(ob, oh.at[pl.ds(0, d), pl.ds(base, BPW)])

    stage_packed(i_interest, 2)
    do_packed(t_interest, 32, 3, ob32, o_interest)
    stage_packed(i_country, 3)
    do_packed(t_country, 16, 7, ob16, o_country)
    stage_packed(i_device, 3)
    do_packed(t_device, 16, 7, ob16, o_device)


def _sc_gather(tables, idxs):
    mesh = plsc.VectorSubcoreMesh(core_axis_name="c", subcore_axis_name="s")
    cp = pltpu.CompilerParams(needs_layout_passes=False)
    user_fn = pl.kernel(
        _sc_user_body,
        out_type=[jax.ShapeDtypeStruct((32, B), jnp.float32)],
        mesh=mesh,
        scratch_types=[pltpu.VMEM((BPW,), jnp.int32),
                       pltpu.VMEM((32, GRP * 128), jnp.float32),
                       pltpu.VMEM((32, BPW), jnp.float32),
                       pltpu.SemaphoreType.DMA],
        compiler_params=cp)
    packed_fn = pl.kernel(
        _sc_packed_body,
        out_type=[jax.ShapeDtypeStruct((16, B), jnp.float32),
                  jax.ShapeDtypeStruct((16, B), jnp.float32),
                  jax.ShapeDtypeStruct((32, B), jnp.float32)],
        mesh=mesh,
        scratch_types=[pltpu.VMEM((BPW,), jnp.int32),
                       pltpu.VMEM((BPW,), jnp.int32),
                       pltpu.VMEM((BPW, 128), jnp.float32),
                       pltpu.VMEM((32, BPW), jnp.float32),
                       pltpu.VMEM((16, BPW), jnp.float32),
                       pltpu.SemaphoreType.DMA],
        compiler_params=cp)
    (euT,) = user_fn(tables[0], idxs[0])
    ecT, edT, eiT = packed_fn(tables[1], tables[2], tables[3],
                              idxs[1], idxs[2], idxs[3])
    return euT, ecT, edT, eiT


def _mlp_body(euT, ecT, edT, eiT, nmT,
              W1r, b1r, g1r, bt1r, W2r, b2r, g2r, bt2r, W3r, b3r,
              out_ref, h1s, h2s, s1, q1, s2, q2):
    f32 = jnp.float32
    p = pl.program_id(0)
    c = pl.program_id(1)
    sl = pl.ds(c * CHUNK, CHUNK)
    dn = (((0,), (0,)), ((), ()))

    @pl.when(p == 0)
    def _phase0():
        h = (lax.dot_general(euT[...], W1r[0:32, :], dn,
                             preferred_element_type=f32)
             + lax.dot_general(ecT[...], W1r[32:48, :], dn,
                               preferred_element_type=f32)
             + lax.dot_general(edT[...], W1r[48:64, :], dn,
                               preferred_element_type=f32)
             + lax.dot_general(eiT[...], W1r[64:96, :], dn,
                               preferred_element_type=f32)
             + lax.dot_general(nmT[...], W1r[96:112, :], dn,
                               preferred_element_type=f32)
             + b1r[...])
        h = jnp.maximum(h, 0.0)
        h1s[sl, :] = h
        hs = jnp.sum(h, axis=0, keepdims=True)
        hq = jnp.sum(h * h, axis=0, keepdims=True)

        @pl.when(c == 0)
        def _():
            s1[...] = hs
            q1[...] = hq

        @pl.when(c != 0)
        def _():
            s1[...] += hs
            q1[...] += hq

    @pl.when(p == 1)
    def _phase1():
        m1 = s1[...] * (1.0 / B)
        v1 = jnp.maximum(q1[...] * (1.0 / B) - m1 * m1, 0.0)
        a1 = lax.rsqrt(v1 + EPS) * g1r[...]
        c1 = bt1r[...] - m1 * a1
        hn = h1s[sl, :] * a1 + c1
        h2 = jnp.maximum(jnp.dot(hn, W2r[...], preferred_element_type=f32)
                         + b2r[...], 0.0)
        h2s[sl, :] = h2
        hs = jnp.sum(h2, axis=0, keepdims=True)
        hq = jnp.sum(h2 * h2, axis=0, keepdims=True)

        @pl.when(c == 0)
        def _():
            s2[...] = hs
            q2[...] = hq

        @pl.when(c != 0)
        def _():
            s2[...] += hs
            q2[...] += hq

    @pl.when(p == 2)
    def _phase2():
        m2 = s2[...] * (1.0 / B)
        v2 = jnp.maximum(q2[...] * (1.0 / B) - m2 * m2, 0.0)
        a2 = lax.rsqrt(v2 + EPS) * g2r[...]
        c2 = bt2r[...] - m2 * a2
        hn = h2s[sl, :] * a2 + c2
        out_ref[...] = (lax.dot_general(W3r[...], hn, (((0,), (1,)), ((), ())),
                                        preferred_element_type=f32)
                        + b3r[...])


def _mlp_tc(euT, ecT, edT, eiT, nmT, W1, b1, g1, bt1, W2, b2, g2, bt2,
            W3, b3):
    def chunk_specT(d):
        return pl.BlockSpec((d, CHUNK),
                            lambda p, c: (0, jnp.where(p == 0, c, 0)))

    def full_spec(shape):
        return pl.BlockSpec(shape, lambda p, c: (0, 0))

    return pl.pallas_call(
        _mlp_body,
        grid=(3, NCHUNK),
        in_specs=[
            chunk_specT(32), chunk_specT(16), chunk_specT(16),
            chunk_specT(32), chunk_specT(16),
            full_spec((112, 256)), full_spec((1, 256)), full_spec((1, 256)),
            full_spec((1, 256)),
            full_spec((256, 128)), full_spec((1, 128)), full_spec((1, 128)),
            full_spec((1, 128)),
            full_spec((128, 64)), full_spec((64, 1)),
        ],
        out_specs=pl.BlockSpec((64, CHUNK),
                               lambda p, c: (0, jnp.where(p == 2, c, 0))),
        out_shape=jax.ShapeDtypeStruct((64, B), jnp.float32),
        scratch_shapes=[
            pltpu.VMEM((B, 256), jnp.float32),
            pltpu.VMEM((B, 128), jnp.float32),
            pltpu.VMEM((1, 256), jnp.float32),
            pltpu.VMEM((1, 256), jnp.float32),
            pltpu.VMEM((1, 128), jnp.float32),
            pltpu.VMEM((1, 128), jnp.float32),
        ],
    )(euT, ecT, edT, eiT, nmT, W1, b1, g1, bt1, W2, b2, g2, bt2, W3, b3)


def kernel(cat_user_id, cat_country, cat_device, cat_interest,
           numerical_inputs,
           T_user, T_country, T_device, T_interest,
           W1, b1, g1, bt1, W2, b2, g2, bt2, W3, b3):
    idxs = (cat_user_id[:, 0], cat_country[:, 0], cat_device[:, 0],
            cat_interest[:, 0])
    tables = (T_user.T, T_country.reshape(-1, 128),
              T_device.reshape(-1, 128), T_interest.reshape(-1, 128))
    euT, ecT, edT, eiT = _sc_gather(tables, idxs)
    outT = _mlp_tc(euT, ecT, edT, eiT, numerical_inputs.T,
                   W1, b1.reshape(1, 256), g1.reshape(1, 256),
                   bt1.reshape(1, 256), W2, b2.reshape(1, 128),
                   g2.reshape(1, 128), bt2.reshape(1, 128),
                   W3, b3.reshape(64, 1))
    return outT.T
